# baseline (speedup 1.0000x reference)
import functools

import jax
import jax.numpy as jnp
from jax import lax
from jax.experimental import pallas as pl
from jax.experimental.pallas import tpu as pltpu
from jax.experimental.pallas import tpu_sc as plsc

_D = 32
_B = 16384
_L = 16
_V = 1000001

_info = plsc.get_sparse_core_info()
_NC, _NS = _info.num_cores, _info.num_subcores
_NW = _NC * _NS
_C = 512
_NCHUNK = 1954
_TPW = 62
_BINCAP = _B + _TPW * (_L - 1) + _L
_NSLOT = 4
_OW = 128
_TRASH = _B

_mesh = plsc.VectorSubcoreMesh(core_axis_name="c", subcore_axis_name="s")


@functools.partial(
    pl.kernel,
    mesh=_mesh,
    out_type=jax.ShapeDtypeStruct((_B + 1, _OW), jnp.float32),
    scratch_types=[
        pltpu.VMEM((_B,), jnp.int32),
        pltpu.VMEM((_B + _L,), jnp.int32),
        pltpu.VMEM((_B + _L,), jnp.int32),
        pltpu.VMEM((_BINCAP,), jnp.int32),
        pltpu.VMEM((_BINCAP,), jnp.int32),
        pltpu.VMEM((64,), jnp.int32),
        pltpu.VMEM((64,), jnp.int32),
        pltpu.VMEM((2, _D, _C), jnp.float32),
        pltpu.VMEM((_NSLOT, _L, _OW), jnp.float32),
        pltpu.SMEM((64,), jnp.int32),
        pltpu.SMEM((64,), jnp.int32),
        pltpu.SemaphoreType.DMA,
        pltpu.SemaphoreType.DMA,
        [pltpu.SemaphoreType.DMA] * _NSLOT,
    ],
    compiler_params=pltpu.CompilerParams(needs_layout_passes=False),
)
def _gather_kernel(
    table_t, idx_hbm, out_hbm,
    idx_all, sel_j, sel_pos, bin_j, bin_pos, hist, run, bufs, ext,
    starts_s, cnt_s, fsem0, fsem1, ssems,
):
    wid = lax.axis_index("s") * _NC + lax.axis_index("c")
    iota = lax.iota(jnp.int32, _L)
    zeros = jnp.zeros((_L,), jnp.int32)
    ones = jnp.ones((_L,), jnp.int32)

    pltpu.sync_copy(idx_hbm, idx_all)
    for v in range(4):
        hist[pl.ds(v * _L, _L)] = zeros

    for k in range(_NSLOT):
        trash = jnp.full((_L,), _TRASH, jnp.int32)
        pltpu.async_copy(ext.at[k], out_hbm.at[trash], ssems[k])

    def swait(k):
        pltpu.make_async_copy(out_hbm.at[pl.ds(0, _L)], ext.at[k], ssems[k]).wait()

    wid_v = jnp.full((_L,), wid, jnp.int32)

    def sel_body(g, cnt):
        jv = idx_all[pl.ds(g * _L, _L)]
        posv = iota + g * _L
        m = ((jv >> 9) & 31) == wid_v
        plsc.store_compressed(sel_j.at[pl.ds(cnt, _L)], jv, mask=m)
        plsc.store_compressed(sel_pos.at[pl.ds(cnt, _L)], posv, mask=m)
        npop = plsc.all_reduce_population_count(m)
        return cnt + npop[0]

    n_sel = lax.fori_loop(0, _B // _L, sel_body, jnp.int32(0), unroll=False)
    n_grp = (n_sel + _L - 1) // _L

    def hist_body(g, _):
        jv = sel_j[pl.ds(g * _L, _L)]
        m = iota < jnp.full((_L,), n_sel - g * _L, jnp.int32)
        t = (jv >> 14) & 63
        plsc.addupdate_scatter(hist, [t], ones, mask=m)
        return 0

    lax.fori_loop(0, n_grp, hist_body, 0, unroll=False)

    carry = jnp.int32(0)
    for v in range(4):
        h = hist[pl.ds(v * _L, _L)]
        p = (h + (_L - 1)) & ~(_L - 1)
        cs = plsc.cumsum(p)
        s = cs - p + carry
        run[pl.ds(v * _L, _L)] = s
        for l in range(_L):
            starts_s[v * _L + l] = s[l]
            cnt_s[v * _L + l] = h[l]
        carry = carry + cs[_L - 1]

    def place_body(g, _):
        jv = sel_j[pl.ds(g * _L, _L)]
        posv = sel_pos[pl.ds(g * _L, _L)]
        m = iota < jnp.full((_L,), n_sel - g * _L, jnp.int32)
        t = (jv >> 14) & 63
        cnt, last = plsc.scan_count(t, m)
        base = plsc.load_gather(run, [t], mask=m)
        dst = base + cnt - 1
        plsc.store_scatter(bin_j, [dst], jv, mask=m)
        plsc.store_scatter(bin_pos, [dst], posv, mask=m)
        plsc.addupdate_scatter(run, [t], cnt, mask=m & last)
        return 0

    lax.fori_loop(0, n_grp, place_body, 0, unroll=False)

    fsems = (fsem0, fsem1)

    def fetch(t, b):
        off = (wid + 32 * t) * _C
        pltpu.async_copy(table_t.at[:, pl.ds(off, _C)], bufs.at[b], fsems[b])

    def fwait(b):
        pltpu.make_async_copy(table_t.at[:, pl.ds(0, _C)], bufs.at[b], fsems[b]).wait()

    def extract_bucket(t, b, base_col):
        s0 = starts_s[t]
        n = cnt_s[t]
        base_v = jnp.full((_L,), base_col, jnp.int32)

        def group(g, k):
            @pl.when(g * _L < n)
            def _():
                jv = bin_j[pl.ds(s0 + g * _L, _L)]
                posv = bin_pos[pl.ds(s0 + g * _L, _L)]
                m = iota < jnp.full((_L,), n - g * _L, jnp.int32)
                lc = jv - base_v
                for f in range(_D):
                    fv = jnp.full((_L,), f, jnp.int32)
                    vals = plsc.load_gather(bufs.at[b], [fv, lc], mask=m)
                    plsc.store_scatter(ext.at[k], [iota, fv], vals)
                pos_pad = jnp.where(m, posv, _TRASH)
                del pos_pad
                swait(k)
                pltpu.async_copy(ext.at[k], out_hbm.at[pl.ds(_TRASH - _L, _L)], ssems[k])

        def gbody(q, _):
            for k in range(_NSLOT):
                group(q * _NSLOT + k, k)
            return 0

        lax.fori_loop(0, (n + _L * _NSLOT - 1) // (_L * _NSLOT), gbody, 0,
                      unroll=False)

    def pair_body(q, _):
        t0 = 2 * q
        fwait(0)
        extract_bucket(t0, 0, (wid + 32 * t0) * _C)

        @pl.when(t0 + 2 < 61)
        def _():
            fetch(t0 + 2, 0)

        t1 = 2 * q + 1
        fwait(1)
        extract_bucket(t1, 1, (wid + 32 * t1) * _C)

        @pl.when(t1 + 2 < 61)
        def _():
            fetch(t1 + 2, 1)

        return 0

    fetch(0, 0)
    fetch(1, 1)
    lax.fori_loop(0, 30, pair_body, 0, unroll=False)
    fwait(0)
    extract_bucket(60, 0, (wid + 32 * 60) * _C)

    @pl.when(wid == 0)
    def _():
        pltpu.async_copy(table_t.at[:, pl.ds(999424, _C)], bufs.at[1], fsem1)
        fwait(1)
        extract_bucket(61, 1, 999424)

    @pl.when(wid == 1)
    def _():
        toff = pl.multiple_of((wid - 1) + 999936, 128)
        pltpu.async_copy(
            table_t.at[:, pl.ds(toff, 128)],
            bufs.at[1].at[:, pl.ds(0, 128)],
            fsem1,
        )
        pltpu.make_async_copy(
            table_t.at[:, pl.ds(0, 128)], bufs.at[1].at[:, pl.ds(0, 128)], fsem1
        ).wait()
        extract_bucket(61, 1, 999936)

    for k in range(_NSLOT):
        swait(k)


def kernel(inputs, table):
    out = _gather_kernel(table.T, inputs.astype(jnp.int32))
    return out[:_B, :_D]


# speedup vs baseline: 2.4634x; 2.4634x over previous
import functools

import jax
import jax.numpy as jnp
from jax import lax
from jax.experimental import pallas as pl
from jax.experimental.pallas import tpu as pltpu
from jax.experimental.pallas import tpu_sc as plsc

_D = 32
_B = 16384
_L = 16
_R = 24
_M = 48

_info = plsc.get_sparse_core_info()
_NC, _NS = _info.num_cores, _info.num_subcores
_NW = _NC * _NS
_BPW = _B // _NW

_mesh = plsc.VectorSubcoreMesh(core_axis_name="c", subcore_axis_name="s")


@functools.partial(
    pl.kernel,
    mesh=_mesh,
    out_type=jax.ShapeDtypeStruct((_D, _B), jnp.float32),
    scratch_types=[
        pltpu.VMEM((_BPW,), jnp.int32),
        pltpu.VMEM((_R, _D, 128), jnp.float32),
        pltpu.VMEM((_D, _BPW), jnp.float32),
        [pltpu.SemaphoreType.DMA] * _R,
    ],
    compiler_params=pltpu.CompilerParams(needs_layout_passes=False),
)
def _gather_kernel(table_t, idx_hbm, out_t, idx_v, ring, slab, sems):
    wid = lax.axis_index("s") * _NC + lax.axis_index("c")
    base = wid * _BPW
    pltpu.sync_copy(idx_hbm.at[pl.ds(base, _BPW)], idx_v)

    rows = lax.iota(jnp.int32, _L)

    def fetch(j, s):
        jt = pl.multiple_of((j // 128) * 128, 128)
        pltpu.async_copy(table_t.at[:, pl.ds(jt, 128)], ring.at[s], sems[s])

    def extract(jr, i, s):
        pltpu.make_async_copy(
            table_t.at[:, pl.ds(0, 128)], ring.at[s], sems[s]
        ).wait()
        col = jnp.full((_L,), jr, jnp.int32)
        dst_col = jnp.full((_L,), i, jnp.int32)
        for h in range(0, _D, _L):
            vals = plsc.load_gather(ring.at[s], [rows + h, col])
            plsc.store_scatter(slab, [rows + h, dst_col], vals)

    vec0 = idx_v[pl.ds(0, _L)]
    vec1 = idx_v[pl.ds(_L, _L)]
    for i in range(_R):
        v = (vec0, vec1)[i // _L]
        fetch(v[i % _L], i)

    def body(k, carry):
        va, vb = carry
        v2 = idx_v[pl.ds((3 * k + 2) * _L, _L)]
        v3 = idx_v[pl.ds((3 * k + 3) * _L, _L)]
        v4 = idx_v[pl.ds((3 * k + 4) * _L, _L)]
        vecs = (va, vb, v2, v3, v4)
        jra, jrb, jr2 = va % 128, vb % 128, v2 % 128
        jrs = (jra, jrb, jr2)
        i0 = _M * k
        for j in range(_M):
            s = j % _R
            extract(jrs[j // _L][j % _L], i0 + j, s)
            fv = vecs[(j + _R) // _L]
            fetch(fv[(j + _R) % _L], s)
        return (v3, v4)

    v30, v31 = lax.fori_loop(0, (_BPW - 2 * _L) // _M, body, (vec0, vec1),
                             unroll=False)

    jr30, jr31 = v30 % 128, v31 % 128
    tail0 = _BPW - 2 * _L
    for j in range(2 * _L):
        i = tail0 + j
        extract((jr30, jr31)[j // _L][j % _L], i, i % _R)
        if i + _R < _BPW:
            v = (v30, v31)[(j + _R) // _L]
            fetch(v[(j + _R) % _L], i % _R)

    pltpu.sync_copy(slab, out_t.at[:, pl.ds(base, _BPW)])


def kernel(inputs, table):
    out_t = _gather_kernel(table.T, inputs.astype(jnp.int32))
    return out_t.T


# speedup vs baseline: 2.5047x; 1.0168x over previous
import functools

import jax
import jax.numpy as jnp
from jax import lax
from jax.experimental import pallas as pl
from jax.experimental.pallas import tpu as pltpu
from jax.experimental.pallas import tpu_sc as plsc

_D = 32
_B = 16384
_L = 16

_info = plsc.get_sparse_core_info()
_NC, _NS = _info.num_cores, _info.num_subcores
_NW = _NC * _NS
_BPW = _B // _NW
_G = _BPW // _L

_mesh = plsc.VectorSubcoreMesh(core_axis_name="c", subcore_axis_name="s")


@functools.partial(
    pl.kernel,
    mesh=_mesh,
    out_type=jax.ShapeDtypeStruct((_D, _B), jnp.float32),
    scratch_types=[
        pltpu.VMEM((_BPW,), jnp.int32),
        pltpu.VMEM((_L, _D, 128), jnp.float32),
        pltpu.VMEM((_D, _BPW), jnp.float32),
        [pltpu.SemaphoreType.DMA] * _L,
    ],
    compiler_params=pltpu.CompilerParams(needs_layout_passes=False),
)
def _gather_kernel(table_t, idx_hbm, out_t, idx_v, ring, slab, sems):
    wid = lax.axis_index("s") * _NC + lax.axis_index("c")
    base = wid * _BPW
    pltpu.sync_copy(idx_hbm.at[pl.ds(base, _BPW)], idx_v)

    rows = lax.iota(jnp.int32, _L)

    def fetch(j, l):
        jt = pl.multiple_of((j // 128) * 128, 128)
        pltpu.async_copy(table_t.at[:, pl.ds(jt, 128)], ring.at[l], sems[l])

    def extract(jr, i, l):
        pltpu.make_async_copy(
            table_t.at[:, pl.ds(0, 128)], ring.at[l], sems[l]
        ).wait()
        col = jnp.full((_L,), jr, jnp.int32)
        dst_col = jnp.full((_L,), i, jnp.int32)
        for h in range(0, _D, _L):
            vals = plsc.load_gather(ring.at[l], [rows + h, col])
            plsc.store_scatter(slab, [rows + h, dst_col], vals)

    vec0 = idx_v[pl.ds(0, _L)]
    for l in range(_L):
        fetch(vec0[l], l)

    def body(g, vec_prev):
        vec = idx_v[pl.ds(g * _L, _L)]
        jr_prev = vec_prev % 128
        for l in range(_L):
            extract(jr_prev[l], (g - 1) * _L + l, l)
            fetch(vec[l], l)
        return vec

    vec_last = lax.fori_loop(1, _G, body, vec0, unroll=False)
    jr_last = vec_last % 128
    for l in range(_L):
        extract(jr_last[l], (_G - 1) * _L + l, l)

    pltpu.sync_copy(slab, out_t.at[:, pl.ds(base, _BPW)])


def kernel(inputs, table):
    out_t = _gather_kernel(table.T, inputs.astype(jnp.int32))
    return out_t.T
